# Initial kernel scaffold; baseline (speedup 1.0000x reference)
import jax
import jax.numpy as jnp
from jax.experimental import pallas as pl


def kernel(scores, coords, batch_ids):
    raise NotImplementedError("write your pallas kernel here")


# speedup vs baseline: 10.9136x; 10.9136x over previous
import jax
import jax.numpy as jnp
from jax.experimental import pallas as pl
from jax.experimental.pallas import tpu as pltpu
from functools import partial

_N = 8192
_R = 128
_K_NEAR = 8
_K_FAR = 16
_TEMP_LOC = 0.1
_TEMP_CON = 0.5


def _block_body(sc_col, sc_row, bid_col, bid_row, crd_blk, crd_t,
                o_wsd, o_wsum, o_lpos, o_lneg, o_dtot, o_dcnt, o_smooth,
                d2_ref):
    i = pl.program_id(0)

    row_sc = sc_col[...]
    all_sc = sc_row[...]
    row_b = bid_col[...]
    all_b = bid_row[...]
    cb = crd_blk[...]
    ct = crd_t[...]

    d2 = ((cb[:, 0:1] - ct[0:1, :]) ** 2
          + (cb[:, 1:2] - ct[1:2, :]) ** 2
          + (cb[:, 2:3] - ct[2:3, :]) ** 2)
    same = row_b == all_b
    d2_ref[...] = jnp.where(same, d2, jnp.inf)

    jidx = jax.lax.broadcasted_iota(jnp.int32, (_R, _N), 1)
    ridx = i * _R + jax.lax.broadcasted_iota(jnp.int32, (_R, 1), 0)

    less = (all_sc < row_sc) | ((all_sc == row_sc) & (jidx < ridx))
    rank = jnp.sum(jnp.where(same & less, 1.0, 0.0), axis=1, keepdims=True)
    n_b = jnp.sum(jnp.where(same, 1.0, 0.0), axis=1, keepdims=True)
    step = 1.0 / (n_b - 1.0)
    sqe = (row_sc - rank * step) ** 2
    ok = n_b >= 2.0
    dtot = jnp.sum(jnp.where(ok, sqe / n_b, 0.0), axis=(0, 1), keepdims=True)
    dcnt = jnp.sum(jnp.where(ok, 1.0 / n_b, 0.0), axis=(0, 1), keepdims=True)

    zero11 = jnp.zeros((1, 1), jnp.float32)
    wsum = zero11
    wsd = zero11
    lpos = zero11
    lneg = zero11
    near_sum = jnp.zeros((_R, 1), jnp.float32)
    for k in range(_K_FAR):
        d2c = d2_ref[...]
        m = jnp.min(d2c, axis=1, keepdims=True)
        idxm = jnp.min(jnp.where(d2c == m, jidx, _N), axis=1,
                       keepdims=True)
        sel = jidx == idxm
        sc_k = jnp.sum(jnp.where(sel, all_sc, 0.0), axis=1,
                       keepdims=True)
        d2_ref[...] = jnp.where(sel, jnp.inf, d2c)
        sd = row_sc - sc_k
        sim = 1.0 - jnp.abs(sd)
        if k < _K_NEAR:
            dist = jnp.sqrt(jnp.maximum(m, 0.0))
            w = jnp.exp(-dist / _TEMP_LOC)
            wsum = wsum + jnp.sum(w, axis=(0, 1), keepdims=True)
            wsd = wsd + jnp.sum(w * sd * sd, axis=(0, 1), keepdims=True)
            near_sum = near_sum + sc_k
            lpos = lpos + jnp.sum(
                jnp.log(jax.nn.sigmoid(sim / _TEMP_CON) + 1e-8),
                axis=(0, 1), keepdims=True)
        else:
            lneg = lneg + jnp.sum(
                jnp.log(1.0 - jax.nn.sigmoid(sim / _TEMP_CON) + 1e-8),
                axis=(0, 1), keepdims=True)

    smooth = jnp.sum((row_sc - near_sum * (1.0 / _K_NEAR)) ** 2,
                     axis=(0, 1), keepdims=True)

    @pl.when(i == 0)
    def _init():
        o_wsd[...] = zero11
        o_wsum[...] = zero11
        o_lpos[...] = zero11
        o_lneg[...] = zero11
        o_dtot[...] = zero11
        o_dcnt[...] = zero11
        o_smooth[...] = zero11

    o_wsd[...] += wsd
    o_wsum[...] += wsum
    o_lpos[...] += lpos
    o_lneg[...] += lneg
    o_dtot[...] += dtot
    o_dcnt[...] += dcnt
    o_smooth[...] += smooth


def _run(scores, coords, batch_ids):
    n = scores.shape[0]
    sc_col = scores.reshape(n, 1)
    sc_row = scores.reshape(1, n)
    bid_col = batch_ids.reshape(n, 1)
    bid_row = batch_ids.reshape(1, n)
    crd_t = coords.T

    num_blocks = n // _R
    one = pl.BlockSpec((1, 1), lambda i: (0, 0))
    outs = pl.pallas_call(
        _block_body,
        grid=(num_blocks,),
        in_specs=[
            pl.BlockSpec((_R, 1), lambda i: (i, 0)),
            pl.BlockSpec((1, n), lambda i: (0, 0)),
            pl.BlockSpec((_R, 1), lambda i: (i, 0)),
            pl.BlockSpec((1, n), lambda i: (0, 0)),
            pl.BlockSpec((_R, 3), lambda i: (i, 0)),
            pl.BlockSpec((3, n), lambda i: (0, 0)),
        ],
        out_specs=[one] * 7,
        out_shape=[jax.ShapeDtypeStruct((1, 1), jnp.float32)] * 7,
        scratch_shapes=[pltpu.VMEM((_R, n), jnp.float32)],
    )(sc_col, sc_row, bid_col, bid_row, coords, crd_t)
    return outs


@jax.jit
def kernel(scores, coords, batch_ids):
    n = scores.shape[0]
    wsd, wsum, lpos, lneg, dtot, dcnt, smooth = [o[0, 0] for o in
                                                 _run(scores, coords,
                                                      batch_ids)]
    loss_loc = wsd / jnp.maximum(wsum, 1e-8)
    loss_con = -(lpos + lneg) / (n * _K_NEAR)
    loss_dist = dtot / jnp.maximum(jnp.round(dcnt), 1.0)
    loss_smooth = smooth / n
    return (1.0 * loss_loc + 0.5 * loss_con
            + 0.3 * loss_dist + 0.2 * loss_smooth)
